# baseline (device time: 28351 ns/iter reference)
import jax
import jax.numpy as jnp
from jax import lax
from jax.experimental import pallas as pl
from jax.experimental.pallas import tpu as pltpu

_BM = 512


def kernel(x, dy, gamma):
    del gamma
    m, d = x.shape
    n_blocks = m // _BM

    def body(x_ref, dy_ref, out_ref, acc_ref, comm_ref, send_sem, recv_sem):
        i = pl.program_id(0)

        my_x = lax.axis_index("x")
        my_y = lax.axis_index("y")

        @pl.when(i == 0)
        def _():
            acc_ref[...] = jnp.zeros_like(acc_ref)
            barrier_sem = pltpu.get_barrier_semaphore()
            pl.semaphore_signal(
                barrier_sem, inc=1,
                device_id=(1 - my_x, my_y),
                device_id_type=pl.DeviceIdType.MESH,
            )

        xb = x_ref[...]
        dyb = dy_ref[...]
        mu = jnp.mean(xb, axis=1, keepdims=True)
        var = jnp.mean((xb - mu) ** 2, axis=1, keepdims=True)
        rstd = lax.rsqrt(var + 1e-5)
        xhat = (xb - mu) * rstd
        dg = jnp.sum(dyb * xhat, axis=0, keepdims=True)
        db = jnp.sum(dyb, axis=0, keepdims=True)
        acc_ref[...] += jnp.concatenate([dg, db], axis=0)

        @pl.when(i == n_blocks - 1)
        def _():
            barrier_sem = pltpu.get_barrier_semaphore()
            pl.semaphore_wait(barrier_sem, 1)

            rdma = pltpu.make_async_remote_copy(
                src_ref=acc_ref,
                dst_ref=comm_ref,
                send_sem=send_sem,
                recv_sem=recv_sem,
                device_id=(1 - my_x, my_y),
                device_id_type=pl.DeviceIdType.MESH,
            )
            rdma.start()
            rdma.wait_recv()
            out_ref[...] = acc_ref[...] + comm_ref[...]
            rdma.wait_send()

    return pl.pallas_call(
        body,
        grid=(n_blocks,),
        out_shape=jax.ShapeDtypeStruct((2, d), jnp.float32),
        in_specs=[
            pl.BlockSpec((_BM, d), lambda i: (i, 0)),
            pl.BlockSpec((_BM, d), lambda i: (i, 0)),
        ],
        out_specs=pl.BlockSpec((2, d), lambda i: (0, 0)),
        scratch_shapes=[
            pltpu.VMEM((2, d), jnp.float32),
            pltpu.VMEM((2, d), jnp.float32),
            pltpu.SemaphoreType.DMA,
            pltpu.SemaphoreType.DMA,
        ],
        compiler_params=pltpu.CompilerParams(
            collective_id=0,
            dimension_semantics=("arbitrary",),
            vmem_limit_bytes=60 * 1024 * 1024,
        ),
    )(x, dy)


# device time: 19439 ns/iter; 1.4585x vs baseline; 1.4585x over previous
import jax
import jax.numpy as jnp
from jax import lax
from jax.experimental import pallas as pl
from jax.experimental.pallas import tpu as pltpu

_BM = 512


def kernel(x, dy, gamma):
    del gamma
    m, d = x.shape
    mh = m // 2
    n_blocks = mh // _BM

    def body(y_ref, x_ref, dy_ref, out_ref, acc_ref, comm_ref,
             send_sems, recv_sems):
        i = pl.program_id(0)
        my_x = lax.axis_index("x")
        my_y = lax.axis_index("y")
        peers = [
            (my_x, 1 - my_y),
            (1 - my_x, my_y),
            (1 - my_x, 1 - my_y),
        ]

        @pl.when(i == 0)
        def _():
            acc_ref[...] = jnp.zeros_like(acc_ref)
            barrier_sem = pltpu.get_barrier_semaphore()
            for tgt in peers:
                pl.semaphore_signal(
                    barrier_sem, inc=1,
                    device_id=tgt,
                    device_id_type=pl.DeviceIdType.MESH,
                )

        xb = x_ref[...]
        dyb = dy_ref[...]
        mu = jnp.mean(xb, axis=1, keepdims=True)
        var = jnp.mean((xb - mu) ** 2, axis=1, keepdims=True)
        rstd = lax.rsqrt(var + 1e-5)
        xhat = (xb - mu) * rstd
        dg = jnp.sum(dyb * xhat, axis=0, keepdims=True)
        db = jnp.sum(dyb, axis=0, keepdims=True)
        acc_ref[...] += jnp.concatenate([dg, db], axis=0)

        @pl.when(i == n_blocks - 1)
        def _():
            barrier_sem = pltpu.get_barrier_semaphore()
            pl.semaphore_wait(barrier_sem, 3)

            rdmas = []
            for j, tgt in enumerate(peers):
                rdma = pltpu.make_async_remote_copy(
                    src_ref=acc_ref,
                    dst_ref=comm_ref.at[j],
                    send_sem=send_sems.at[j],
                    recv_sem=recv_sems.at[j],
                    device_id=tgt,
                    device_id_type=pl.DeviceIdType.MESH,
                )
                rdma.start()
                rdmas.append(rdma)
            for rdma in rdmas:
                rdma.wait_recv()
            out_ref[...] = (
                acc_ref[...] + comm_ref[0] + comm_ref[1] + comm_ref[2]
            )
            for rdma in rdmas:
                rdma.wait_send()

    def in_index(i, y_ref):
        return (y_ref[0] * n_blocks + i, 0)

    grid_spec = pltpu.PrefetchScalarGridSpec(
        num_scalar_prefetch=1,
        grid=(n_blocks,),
        in_specs=[
            pl.BlockSpec((_BM, d), in_index),
            pl.BlockSpec((_BM, d), in_index),
        ],
        out_specs=pl.BlockSpec((2, d), lambda i, y_ref: (0, 0)),
        scratch_shapes=[
            pltpu.VMEM((2, d), jnp.float32),
            pltpu.VMEM((3, 2, d), jnp.float32),
            pltpu.SemaphoreType.DMA((3,)),
            pltpu.SemaphoreType.DMA((3,)),
        ],
    )

    y_idx = lax.axis_index("y").astype(jnp.int32)[None]
    return pl.pallas_call(
        body,
        grid_spec=grid_spec,
        out_shape=jax.ShapeDtypeStruct((2, d), jnp.float32),
        compiler_params=pltpu.CompilerParams(
            collective_id=0,
            dimension_semantics=("arbitrary",),
            vmem_limit_bytes=60 * 1024 * 1024,
        ),
    )(y_idx, x, dy)
